# Initial kernel scaffold; baseline (speedup 1.0000x reference)
import jax
import jax.numpy as jnp
from jax.experimental import pallas as pl


def kernel(v3_out):
    raise NotImplementedError("write your pallas kernel here")


# speedup vs baseline: 11.5411x; 11.5411x over previous
import jax
import jax.numpy as jnp
from jax.experimental import pallas as pl
from jax.experimental.pallas import tpu as pltpu

_CONF = 0.5
_NMS = 0.4
_MAXD = 100
_NCLS = 80
_N = 20000
_NPAD = 20480
_CHUNK = 1280


def _prep_body(p_ref, score_ref, x1_ref, y1_ref, x2_ref, y2_ref, cls_ref):
    cx = p_ref[0:1, :]
    cy = p_ref[1:2, :]
    w = p_ref[2:3, :]
    h = p_ref[3:4, :]
    obj = p_ref[4:5, :]
    cc = p_ref[5:85, :] * obj
    smax = jnp.max(cc, axis=0, keepdims=True)
    ids = jax.lax.broadcasted_iota(jnp.int32, (80, _CHUNK), 0)
    cls_i = jnp.min(jnp.where(cc == smax, ids, _NCLS), axis=0, keepdims=True)
    cls = cls_i.astype(jnp.float32)
    score_ref[...] = jnp.where(smax > _CONF, smax, 0.0)[None]
    off = cls * 4.0
    x1_ref[...] = ((cx - w / 2.0) + off)[None]
    y1_ref[...] = ((cy - h / 2.0) + off)[None]
    x2_ref[...] = ((cx + w / 2.0) + off)[None]
    y2_ref[...] = ((cy + h / 2.0) + off)[None]
    cls_ref[...] = cls[None]


def _nms_body(score_ref, x1_ref, y1_ref, x2_ref, y2_ref, cls_ref, out_ref,
              s_scr):
    s_scr[...] = score_ref[...]
    x1 = x1_ref[...]
    y1 = y1_ref[...]
    x2 = x2_ref[...]
    y2 = y2_ref[...]
    area = (x2 - x1) * (y2 - y1)
    flatidx = (jax.lax.broadcasted_iota(jnp.int32, (16, _CHUNK), 0) * _CHUNK
               + jax.lax.broadcasted_iota(jnp.int32, (16, _CHUNK), 1))
    out_ref[...] = jnp.zeros_like(out_ref)
    orow = jax.lax.broadcasted_iota(jnp.int32, (104, 128), 0)
    ocol = jax.lax.broadcasted_iota(jnp.int32, (104, 128), 1)

    def step(d, _):
        s = s_scr[...]
        gmax = jnp.max(s)
        gidx = jnp.min(jnp.where(s == gmax, flatidx, _NPAD))
        onehot = flatidx == gidx
        x1w = jnp.sum(jnp.where(onehot, x1, 0.0))
        y1w = jnp.sum(jnp.where(onehot, y1, 0.0))
        x2w = jnp.sum(jnp.where(onehot, x2, 0.0))
        y2w = jnp.sum(jnp.where(onehot, y2, 0.0))
        clsw = jnp.sum(jnp.where(onehot, cls_ref[...], 0.0))
        xx1 = jnp.maximum(x1w, x1)
        yy1 = jnp.maximum(y1w, y1)
        xx2 = jnp.minimum(x2w, x2)
        yy2 = jnp.minimum(y2w, y2)
        inter = jnp.clip(xx2 - xx1, 0.0) * jnp.clip(yy2 - yy1, 0.0)
        areaw = (x2w - x1w) * (y2w - y1w)
        iou = inter / (areaw + area - inter + 1e-9)
        s_scr[...] = jnp.where((iou > _NMS) | onehot, 0.0, s)
        valid = (gmax > 0.0).astype(jnp.float32)
        off = clsw * 4.0
        vals = ((x1w - off) * valid, (y1w - off) * valid, (x2w - off) * valid,
                (y2w - off) * valid, gmax * valid, clsw * valid)
        detrow = jnp.zeros((104, 128), jnp.float32)
        for k, v in enumerate(vals):
            detrow = jnp.where(ocol == k, v, detrow)
        out_ref[...] = jnp.where(orow == d, detrow, out_ref[...])
        return ()

    jax.lax.fori_loop(0, _MAXD, step, ())


def kernel(v3_out):
    pred_t = jnp.transpose(v3_out[0])
    pred_t = jnp.pad(pred_t, ((0, 3), (0, _NPAD - _N)))
    f32 = jnp.float32
    prep = pl.pallas_call(
        _prep_body,
        grid=(16,),
        in_specs=[pl.BlockSpec((88, _CHUNK), lambda i: (0, i))],
        out_specs=[pl.BlockSpec((1, 1, _CHUNK), lambda i: (i, 0, 0))] * 6,
        out_shape=[jax.ShapeDtypeStruct((16, 1, _CHUNK), f32)] * 6,
    )
    score, x1, y1, x2, y2, cls = (a.reshape(16, _CHUNK) for a in prep(pred_t))
    dets = pl.pallas_call(
        _nms_body,
        out_shape=jax.ShapeDtypeStruct((104, 128), f32),
        scratch_shapes=[pltpu.VMEM((16, _CHUNK), f32)],
    )(score, x1, y1, x2, y2, cls)
    return jax.lax.stop_gradient(dets[:_MAXD, :6][None])
